# baseline (speedup 1.0000x reference)
import jax
import jax.numpy as jnp
from jax.experimental import pallas as pl

_B, _C, _H, _W = 8, 384, 32, 32
_HW = _H * _W
_TEMP = 0.07
_TOPK = 8
_PAD = 6
_K2 = 13 * 13
_NEG = -1e30
_EXPM10 = 4.5399929762484854e-05
_HG = 4
_RG = _HG * _W
_NG = _H // _HG


def _band(hg):
    c0 = max(0, (_HG * hg - _PAD)) * _W
    c1 = min(_H, _HG * hg + _HG - 1 + _PAD + 1) * _W
    c0 = (c0 // 128) * 128
    c1 = min(_HW, ((c1 + 127) // 128) * 128)
    return c0, c1


def _lw_kernel(q_ref, k_ref, o_ref):
    q = q_ref[0] * (1.0 / _TEMP)
    k = k_ref[0]

    for hg in range(_NG):
        c0, c1 = _band(hg)
        wd = c1 - c0
        r0 = hg * _RG
        att = jax.lax.dot_general(
            q[:, r0:r0 + _RG], k[:, c0:c1], (((0,), (0,)), ((), ())),
            preferred_element_type=jnp.float32,
            precision=jax.lax.Precision.HIGHEST)

        l1 = jax.lax.broadcasted_iota(jnp.int32, (_RG, 1), 0) + r0
        h1 = l1 // _W
        w1 = l1 % _W
        n1 = jax.lax.broadcasted_iota(jnp.int32, (1, wd), 1) + c0
        hp = n1 // _W
        wp = n1 % _W
        dh = (hp - h1 + _PAD).astype(jnp.uint32)
        dw = (wp - w1 + _PAD).astype(jnp.uint32)
        in_win = (dh <= 2 * _PAD) & (dw <= 2 * _PAD)

        rows_in = jnp.minimum(h1, _PAD) + jnp.minimum(_H - 1 - h1, _PAD) + 1
        cols_in = jnp.minimum(w1, _PAD) + jnp.minimum(_W - 1 - w1, _PAD) + 1
        n_oob = _K2 - rows_in * cols_in

        attw = jnp.where(in_win, att, _NEG)

        cur = attw
        cum = jnp.zeros((_RG, 1), jnp.float32)
        t8 = jnp.zeros((_RG, 1), jnp.float32)
        sel = jnp.zeros((_RG, 1), jnp.float32)
        topkf = jnp.float32(_TOPK)
        want = topkf - n_oob.astype(jnp.float32)
        for _ in range(_TOPK):
            m = jnp.max(cur, axis=-1, keepdims=True)
            eq = cur == m
            cnt = jnp.sum(jnp.where(eq, 1.0, 0.0), axis=-1, keepdims=True)
            ncum = cum + cnt
            t8 = jnp.where((cum < topkf) & (ncum >= topkf), m, t8)
            sel = jnp.where((cum < want) & (ncum >= want), m, sel)
            cum = ncum
            cur = jnp.where(eq, _NEG, cur)

        z = jnp.sum(jnp.where(attw >= 0.0, 1.0, 0.0), axis=-1, keepdims=True)
        thresh = jnp.where(t8 >= 0.0, t8,
                           jnp.where(z + n_oob.astype(jnp.float32) >= topkf,
                                     0.0, sel))

        u = (attw >= thresh) & (attw != 0.0)
        e = jnp.exp(jnp.where(u, attw, -10.0))
        out = jnp.where(in_win, e, 0.0)

        corr = n_oob.astype(jnp.float32) * _EXPM10
        parts = []
        if c0 == 0:
            out = out + jnp.where(n1 == 0, corr, 0.0)
        else:
            parts.append(corr)
            if c0 > 1:
                parts.append(jnp.zeros((_RG, c0 - 1), jnp.float32))
        parts.append(out)
        if c1 < _HW:
            parts.append(jnp.zeros((_RG, _HW - c1), jnp.float32))
        full = jnp.concatenate(parts, axis=1) if len(parts) > 1 else parts[0]

        o_ref[0, :, _HG * hg:_HG * hg + _HG, :] = (
            full.T.reshape(_HW, _HG, _W))


@jax.jit
def kernel(query, keys):
    q3 = query.reshape(_B, _C, _HW)
    k3 = keys.reshape(_B, _C, _HW)
    out = pl.pallas_call(
        _lw_kernel,
        grid=(_B,),
        in_specs=[
            pl.BlockSpec((1, _C, _HW), lambda b: (b, 0, 0)),
            pl.BlockSpec((1, _C, _HW), lambda b: (b, 0, 0)),
        ],
        out_specs=pl.BlockSpec((1, _HW, _H, _W), lambda b: (b, 0, 0, 0)),
        out_shape=jax.ShapeDtypeStruct((_B, _HW, _H, _W), jnp.float32),
    )(q3, k3)
    return out


# speedup vs baseline: 1.8897x; 1.8897x over previous
import jax
import jax.numpy as jnp
from jax.experimental import pallas as pl

_B, _C, _H, _W = 8, 384, 32, 32
_HW = _H * _W
_TEMP = 0.07
_TOPK = 8
_PAD = 6
_K2 = 13 * 13
_NEG = -1e30
_EXPM10 = 4.5399929762484854e-05
_HG = 4
_RG = _HG * _W
_NG = _H // _HG


def _band(hg):
    c0 = max(0, (_HG * hg - _PAD)) * _W
    c1 = min(_H, _HG * hg + _HG - 1 + _PAD + 1) * _W
    c0 = (c0 // 128) * 128
    c1 = min(_HW, ((c1 + 127) // 128) * 128)
    return c0, c1


def _lw_kernel(q_ref, k_ref, o_ref):
    q = q_ref[0] * (1.0 / _TEMP)
    k = k_ref[0]

    for hg in range(_NG):
        c0, c1 = _band(hg)
        wd = c1 - c0
        r0 = hg * _RG
        att = jax.lax.dot_general(
            q[:, r0:r0 + _RG], k[:, c0:c1], (((0,), (0,)), ((), ())),
            preferred_element_type=jnp.float32,
            precision=jax.lax.Precision.HIGHEST)

        l1 = jax.lax.broadcasted_iota(jnp.int32, (_RG, 1), 0) + r0
        h1 = l1 // _W
        w1 = l1 % _W
        n1 = jax.lax.broadcasted_iota(jnp.int32, (1, wd), 1) + c0
        hp = n1 // _W
        wp = n1 % _W
        dh = (hp - h1 + _PAD).astype(jnp.uint32)
        dw = (wp - w1 + _PAD).astype(jnp.uint32)
        in_win = (dh <= 2 * _PAD) & (dw <= 2 * _PAD)

        rows_in = jnp.minimum(h1, _PAD) + jnp.minimum(_H - 1 - h1, _PAD) + 1
        cols_in = jnp.minimum(w1, _PAD) + jnp.minimum(_W - 1 - w1, _PAD) + 1
        n_oob = _K2 - rows_in * cols_in

        attw = jnp.where(in_win, att, _NEG)

        cur = attw
        cum = jnp.zeros((_RG, 1), jnp.float32)
        t8 = jnp.zeros((_RG, 1), jnp.float32)
        sel = jnp.zeros((_RG, 1), jnp.float32)
        topkf = jnp.float32(_TOPK)
        want = topkf - n_oob.astype(jnp.float32)
        for _ in range(_TOPK):
            m = jnp.max(cur, axis=-1, keepdims=True)
            eq = cur == m
            cnt = jnp.sum(jnp.where(eq, 1.0, 0.0), axis=-1, keepdims=True)
            ncum = cum + cnt
            t8 = jnp.where((cum < topkf) & (ncum >= topkf), m, t8)
            sel = jnp.where((cum < want) & (ncum >= want), m, sel)
            cum = ncum
            cur = jnp.where(eq, _NEG, cur)

        z = jnp.sum(jnp.where(attw >= 0.0, 1.0, 0.0), axis=-1, keepdims=True)
        thresh = jnp.where(t8 >= 0.0, t8,
                           jnp.where(z + n_oob.astype(jnp.float32) >= topkf,
                                     0.0, sel))

        u = (attw >= thresh) & (attw != 0.0)
        e = jnp.exp(jnp.where(u, attw, -10.0))
        out = jnp.where(in_win, e, 0.0)

        corr = n_oob.astype(jnp.float32) * _EXPM10
        parts = []
        if c0 == 0:
            out = out + jnp.where(n1 == 0, corr, 0.0)
        else:
            parts.append(corr)
            if c0 > 1:
                parts.append(jnp.zeros((_RG, c0 - 1), jnp.float32))
        parts.append(out)
        if c1 < _HW:
            parts.append(jnp.zeros((_RG, _HW - c1), jnp.float32))
        full = jnp.concatenate(parts, axis=1) if len(parts) > 1 else parts[0]

        o_ref[0, :, r0:r0 + _RG] = full.T


@jax.jit
def kernel(query, keys):
    q3 = query.reshape(_B, _C, _HW)
    k3 = keys.reshape(_B, _C, _HW)
    out = pl.pallas_call(
        _lw_kernel,
        grid=(_B,),
        in_specs=[
            pl.BlockSpec((1, _C, _HW), lambda b: (b, 0, 0)),
            pl.BlockSpec((1, _C, _HW), lambda b: (b, 0, 0)),
        ],
        out_specs=pl.BlockSpec((1, _HW, _HW), lambda b: (b, 0, 0)),
        out_shape=jax.ShapeDtypeStruct((_B, _HW, _HW), jnp.float32),
    )(q3, k3)
    return out.reshape(_B, _HW, _H, _W)
